# baseline (speedup 1.0000x reference)
import functools

import jax
import jax.numpy as jnp
from jax import lax
from jax.experimental import pallas as pl
from jax.experimental.pallas import tpu as pltpu
from jax.experimental.pallas import tpu_sc as plsc

VOCAB = 1000000
D = 64
B = 4096
S = 200
NC = 2
NS = 16
NW = NC * NS
BPW = B // NW
CB = 2
NCHUNK = BPW // CB
GSZ = 40
NGB = S // GSZ
NV = D // 16


def _sc_body(idx_hbm, table_hbm, pos_hbm, out_hbm,
             idx_v, rows0, rows1, pos_v, sem0, sem1):
    c = lax.axis_index("c")
    s = lax.axis_index("s")
    wid = s * NC + c
    base = wid * BPW
    pltpu.sync_copy(pos_hbm, pos_v)
    pltpu.sync_copy(idx_hbm.at[pl.ds(pl.multiple_of(base, BPW), BPW)], idx_v)

    def start(rows, sem, g):
        b0 = g * CB
        for i in range(CB):
            for j in range(NGB):
                pltpu.async_copy(
                    table_hbm.at[idx_v.at[b0 + i, pl.ds(j * GSZ, GSZ)]],
                    rows.at[i, pl.ds(j * GSZ, GSZ)],
                    sem,
                )

    def finish(rows, sem, g):
        pltpu.make_async_copy(out_hbm.at[pl.ds(0, CB)], rows, sem).wait()

        def add_body(p):
            pv = [pos_v[p, pl.ds(16 * k, 16)] for k in range(NV)]
            for i in range(CB):
                for k in range(NV):
                    rows[i, p, pl.ds(16 * k, 16)] = (
                        rows[i, p, pl.ds(16 * k, 16)] + pv[k]
                    )

        plsc.parallel_loop(0, S, unroll=4)(add_body)
        b0 = pl.multiple_of(base + g * CB, CB)
        for i in range(CB):
            pltpu.sync_copy(rows.at[i], out_hbm.at[b0 + i])

    start(rows0, sem0, 0)

    def pair_body(h, carry):
        g0 = 2 * h
        start(rows1, sem1, g0 + 1)
        finish(rows0, sem0, g0)
        start(rows0, sem0, g0 + 2)
        finish(rows1, sem1, g0 + 1)
        return carry

    lax.fori_loop(0, NCHUNK // 2 - 1, pair_body, 0)
    start(rows1, sem1, NCHUNK - 1)
    finish(rows0, sem0, NCHUNK - 2)
    finish(rows1, sem1, NCHUNK - 1)


@jax.jit
def _run(idx, table, pos2d):
    mesh = plsc.VectorSubcoreMesh(core_axis_name="c", subcore_axis_name="s")
    f = functools.partial(
        pl.kernel,
        out_type=jax.ShapeDtypeStruct((B, S, D), jnp.float32),
        mesh=mesh,
        scratch_types=[
            pltpu.VMEM((BPW, S), jnp.int32),
            pltpu.VMEM((CB, S, D), jnp.float32),
            pltpu.VMEM((CB, S, D), jnp.float32),
            pltpu.VMEM((S, D), jnp.float32),
            pltpu.SemaphoreType.DMA,
            pltpu.SemaphoreType.DMA,
        ],
        compiler_params=pltpu.CompilerParams(use_tc_tiling_on_sc=False),
    )(_sc_body)
    return f(idx, table, pos2d)


def kernel(INPUT, embedding_table, positional_encoding):
    pos2d = positional_encoding[0, :S, :]
    return _run(INPUT, embedding_table, pos2d)
